# baseline (speedup 1.0000x reference)
import functools

import jax
import jax.numpy as jnp
from jax import lax
from jax.experimental import pallas as pl
from jax.experimental.pallas import tpu as pltpu
from jax.experimental.pallas import tpu_sc as plsc

_L = 16
_NUM_FACTORS = 32

_MESH = dict(core_axis_name="c", subcore_axis_name="s")
_PARAMS = dict(needs_layout_passes=False, use_tc_tiling_on_sc=False)


def _build_u(batch, num_rows, num_workers, nc):
    b_per_w = batch // num_workers
    n_groups = b_per_w // _L

    @functools.partial(
        pl.kernel,
        out_type=[
            jax.ShapeDtypeStruct((batch, _NUM_FACTORS), jnp.float32),
            jax.ShapeDtypeStruct((batch,), jnp.float32),
        ],
        mesh=plsc.VectorSubcoreMesh(**_MESH),
        compiler_params=pltpu.CompilerParams(**_PARAMS),
        scratch_types=[
            pltpu.VMEM((b_per_w,), jnp.int32),
            pltpu.VMEM((b_per_w, _NUM_FACTORS), jnp.float32),
            pltpu.VMEM((b_per_w,), jnp.float32),
            pltpu.VMEM((_L,), jnp.float32),
            pltpu.SemaphoreType.DMA,
            pltpu.SemaphoreType.DMA,
        ],
    )
    def u_kernel(uidx_hbm, uf_hbm, ub_hbm, g_hbm, rows_out, bias_out,
                 uidx_v, urow_v, ub_v, g_v, sem_u, sem_ub):
        wid = lax.axis_index("s") * nc + lax.axis_index("c")
        base = wid * b_per_w

        pltpu.sync_copy(uidx_hbm.at[pl.ds(base, b_per_w)], uidx_v)
        pltpu.sync_copy(g_hbm, g_v)
        cp_u = pltpu.async_copy(uf_hbm.at[uidx_v], urow_v, sem_u)
        cp_ub = pltpu.async_copy(ub_hbm.at[uidx_v], ub_v, sem_ub)
        cp_ub.wait()

        gvec = g_v[...]

        def group_body(g, _):
            off = pl.multiple_of(g * _L, _L)
            ub_v[pl.ds(off, _L)] = ub_v[pl.ds(off, _L)] + gvec
            return _

        lax.fori_loop(0, n_groups, group_body, None)
        pltpu.sync_copy(ub_v, bias_out.at[pl.ds(base, b_per_w)])
        cp_u.wait()
        pltpu.sync_copy(urow_v, rows_out.at[pl.ds(base, b_per_w), :])

    return u_kernel


def _build_p(batch, num_workers, nc):
    b_per_w = batch // num_workers
    n_groups = b_per_w // _L

    @functools.partial(
        pl.kernel,
        out_type=jax.ShapeDtypeStruct((batch,), jnp.float32),
        mesh=plsc.VectorSubcoreMesh(**_MESH),
        compiler_params=pltpu.CompilerParams(**_PARAMS),
        scratch_types=[
            pltpu.VMEM((b_per_w,), jnp.int32),
            pltpu.VMEM((b_per_w, _NUM_FACTORS), jnp.float32),
            pltpu.VMEM((b_per_w, _NUM_FACTORS), jnp.float32),
            pltpu.VMEM((b_per_w,), jnp.float32),
            pltpu.VMEM((b_per_w,), jnp.float32),
            pltpu.VMEM((b_per_w,), jnp.float32),
            pltpu.SemaphoreType.DMA,
            pltpu.SemaphoreType.DMA,
        ],
    )
    def p_kernel(pidx_hbm, pf_hbm, pb_hbm, urows_hbm, ubias_hbm, out_hbm,
                 pidx_v, urow_v, prow_v, pb_v, ub_v, out_v, sem_p, sem_pb):
        wid = lax.axis_index("s") * nc + lax.axis_index("c")
        base = wid * b_per_w

        pltpu.sync_copy(pidx_hbm.at[pl.ds(base, b_per_w)], pidx_v)
        cp_p = pltpu.async_copy(pf_hbm.at[pidx_v], prow_v, sem_p)
        cp_pb = pltpu.async_copy(pb_hbm.at[pidx_v], pb_v, sem_pb)
        pltpu.sync_copy(urows_hbm.at[pl.ds(base, b_per_w), :], urow_v)
        pltpu.sync_copy(ubias_hbm.at[pl.ds(base, b_per_w)], ub_v)
        cp_p.wait()
        cp_pb.wait()

        lanes = lax.iota(jnp.int32, _L)

        def group_body(g, _):
            off = pl.multiple_of(g * _L, _L)
            rows = off + lanes
            acc = jnp.zeros((_L,), jnp.float32)
            for d in range(_NUM_FACTORS):
                cols = jnp.full((_L,), d, jnp.int32)
                u = plsc.load_gather(urow_v, [rows, cols])
                p = plsc.load_gather(prow_v, [rows, cols])
                acc = acc + u * p
            out_v[pl.ds(off, _L)] = acc + ub_v[pl.ds(off, _L)] + pb_v[pl.ds(off, _L)]
            return _

        lax.fori_loop(0, n_groups, group_body, None)
        pltpu.sync_copy(out_v, out_hbm.at[pl.ds(base, b_per_w)])

    return p_kernel


def kernel(user_indices, post_indices, user_factors, post_factors,
           user_intercepts, post_intercepts, global_intercept):
    info = plsc.get_sparse_core_info()
    nc, ns = info.num_cores, info.num_subcores
    batch = user_indices.shape[0]
    num_rows = user_factors.shape[0]
    u_call = _build_u(batch, num_rows, nc * ns, nc)
    p_call = _build_p(batch, nc * ns, nc)
    urows, ubias = u_call(
        user_indices.astype(jnp.int32),
        user_factors,
        user_intercepts.reshape(-1),
        jnp.broadcast_to(global_intercept.astype(jnp.float32), (_L,)),
    )
    return p_call(
        post_indices.astype(jnp.int32),
        post_factors,
        post_intercepts.reshape(-1),
        urows,
        ubias,
    )
